# Initial kernel scaffold; baseline (speedup 1.0000x reference)
import jax
import jax.numpy as jnp
from jax.experimental import pallas as pl


def kernel(x, edge_index, W1, b1, W2, b2):
    raise NotImplementedError("write your pallas kernel here")


# speedup vs baseline: 21.7800x; 21.7800x over previous
import functools

import jax
import jax.numpy as jnp
from jax import lax
from jax.experimental import pallas as pl
from jax.experimental.pallas import tpu as pltpu
from jax.experimental.pallas import tpu_sc as plsc

N = 10000
E = 320000
D_IN = 128
D_HID = 16
D_OUT = 128

NC = 2
NS = 16
NW = NC * NS
C = 128
NCHUNK = -(-E // (NW * C))
EPT = NCHUNK * C
E_PAD = EPT * NW
N_P = 10240
RPT = N_P // NS

_f32 = jnp.float32
_i32 = jnp.int32

_mesh = plsc.VectorSubcoreMesh(core_axis_name="c", subcore_axis_name="s")
_sc_params = pltpu.CompilerParams(use_tc_tiling_on_sc=False)


@functools.partial(
    pl.kernel,
    out_type=jax.ShapeDtypeStruct((NC * N_P, D_HID), _f32),
    mesh=_mesh,
    scratch_types=[
        pltpu.VMEM((C,), _i32),
        pltpu.VMEM((C,), _i32),
        pltpu.VMEM((C, D_HID), _f32),
        pltpu.VMEM_SHARED((N_P, D_HID), _f32),
    ],
    compiler_params=_sc_params,
)
def _seg_sum(table_hbm, src_hbm, dst_hbm, zeros_hbm, out_hbm,
             src_v, dst_v, rows_v, acc_sh):
    cid = lax.axis_index("c")
    sid = lax.axis_index("s")
    tid = cid * NS + sid
    pltpu.sync_copy(zeros_hbm.at[pl.ds(sid * RPT, RPT)],
                    acc_sh.at[pl.ds(sid * RPT, RPT)])
    plsc.subcore_barrier()
    base = tid * EPT

    @pl.loop(0, NCHUNK)
    def _(ci):
        off = base + ci * C
        pltpu.sync_copy(src_hbm.at[pl.ds(off, C)], src_v)
        pltpu.sync_copy(dst_hbm.at[pl.ds(off, C)], dst_v)
        pltpu.sync_copy(table_hbm.at[src_v], rows_v)
        pltpu.sync_copy(rows_v, acc_sh.at[dst_v], add=True)

    plsc.subcore_barrier()
    pltpu.sync_copy(acc_sh.at[pl.ds(sid * RPT, RPT)],
                    out_hbm.at[pl.ds(cid * N_P + sid * RPT, RPT)])


@functools.partial(
    pl.kernel,
    out_type=jax.ShapeDtypeStruct((NC * N_P, D_HID), _f32),
    mesh=_mesh,
    scratch_types=[
        pltpu.VMEM((C,), _i32),
        pltpu.VMEM((C, D_HID), _f32),
        pltpu.VMEM_SHARED((N_P, D_HID), _f32),
    ],
    compiler_params=_sc_params,
)
def _count(dst_hbm, zeros_hbm, ones_hbm, out_hbm, dst_v, rows_v, acc_sh):
    cid = lax.axis_index("c")
    sid = lax.axis_index("s")
    tid = cid * NS + sid
    pltpu.sync_copy(zeros_hbm.at[pl.ds(sid * RPT, RPT)],
                    acc_sh.at[pl.ds(sid * RPT, RPT)])
    pltpu.sync_copy(ones_hbm, rows_v)
    plsc.subcore_barrier()
    base = tid * EPT

    @pl.loop(0, NCHUNK)
    def _(ci):
        pltpu.sync_copy(dst_hbm.at[pl.ds(base + ci * C, C)], dst_v)
        pltpu.sync_copy(rows_v, acc_sh.at[dst_v], add=True)

    plsc.subcore_barrier()
    pltpu.sync_copy(acc_sh.at[pl.ds(sid * RPT, RPT)],
                    out_hbm.at[pl.ds(cid * N_P + sid * RPT, RPT)])


BLK = 1024
GRID = N_P // BLK


def _mm1_body(x_ref, w_ref, o_ref):
    o_ref[...] = jnp.dot(x_ref[...], w_ref[...], preferred_element_type=_f32)


def _matmul1(x_p, W1):
    return pl.pallas_call(
        _mm1_body,
        grid=(GRID,),
        in_specs=[pl.BlockSpec((BLK, D_IN), lambda i: (i, 0)),
                  pl.BlockSpec((D_IN, D_HID), lambda i: (0, 0))],
        out_specs=pl.BlockSpec((BLK, D_HID), lambda i: (i, 0)),
        out_shape=jax.ShapeDtypeStruct((N_P, D_HID), _f32),
    )(x_p, W1)


def _scale1_body(cnt_ref, h1_ref, hs_ref, dinv_ref):
    dinv = lax.rsqrt(cnt_ref[0] + cnt_ref[1] + 1.0)
    dinv_ref[...] = dinv
    hs_ref[...] = dinv * h1_ref[...]


def _scale1(cnt, h1):
    return pl.pallas_call(
        _scale1_body,
        grid=(GRID,),
        in_specs=[pl.BlockSpec((NC, BLK, D_HID), lambda i: (0, i, 0)),
                  pl.BlockSpec((BLK, D_HID), lambda i: (i, 0))],
        out_specs=[pl.BlockSpec((BLK, D_HID), lambda i: (i, 0)),
                   pl.BlockSpec((BLK, D_HID), lambda i: (i, 0))],
        out_shape=[jax.ShapeDtypeStruct((N_P, D_HID), _f32),
                   jax.ShapeDtypeStruct((N_P, D_HID), _f32)],
    )(cnt, h1)


def _scale2_body(s1_ref, h1s_ref, dinv_ref, b1_ref, o_ref):
    out1 = dinv_ref[...] * (s1_ref[0] + s1_ref[1] + h1s_ref[...]) + b1_ref[...]
    o_ref[...] = dinv_ref[...] * jnp.maximum(out1, 0.0)


def _scale2(s1, h1s, dinv, b1):
    return pl.pallas_call(
        _scale2_body,
        grid=(GRID,),
        in_specs=[pl.BlockSpec((NC, BLK, D_HID), lambda i: (0, i, 0)),
                  pl.BlockSpec((BLK, D_HID), lambda i: (i, 0)),
                  pl.BlockSpec((BLK, D_HID), lambda i: (i, 0)),
                  pl.BlockSpec((1, D_HID), lambda i: (0, 0))],
        out_specs=pl.BlockSpec((BLK, D_HID), lambda i: (i, 0)),
        out_shape=jax.ShapeDtypeStruct((N_P, D_HID), _f32),
    )(s1, h1s, dinv, b1)


def _final_body(s2_ref, hs_ref, dinv_ref, w2_ref, b2_ref, o_ref):
    u = dinv_ref[...] * (s2_ref[0] + s2_ref[1] + hs_ref[...])
    o_ref[...] = jnp.dot(u, w2_ref[...], preferred_element_type=_f32) + b2_ref[...]


def _final(s2, hs, dinv, W2, b2):
    return pl.pallas_call(
        _final_body,
        grid=(GRID,),
        in_specs=[pl.BlockSpec((NC, BLK, D_HID), lambda i: (0, i, 0)),
                  pl.BlockSpec((BLK, D_HID), lambda i: (i, 0)),
                  pl.BlockSpec((BLK, D_HID), lambda i: (i, 0)),
                  pl.BlockSpec((D_HID, D_OUT), lambda i: (0, 0)),
                  pl.BlockSpec((1, D_OUT), lambda i: (0, 0))],
        out_specs=pl.BlockSpec((BLK, D_OUT), lambda i: (i, 0)),
        out_shape=jax.ShapeDtypeStruct((N_P, D_OUT), _f32),
    )(s2, hs, dinv, W2, b2)


def kernel(x, edge_index, W1, b1, W2, b2):
    x = x.astype(_f32)
    src = edge_index[0].astype(_i32)
    dst = edge_index[1].astype(_i32)
    src_p = jnp.concatenate([src, jnp.zeros((E_PAD - E,), _i32)])
    dst_p = jnp.concatenate([dst, jnp.full((E_PAD - E,), N, _i32)])
    x_p = jnp.pad(x, ((0, N_P - N), (0, 0)))
    zeros = jnp.zeros((N_P, D_HID), _f32)
    ones = jnp.ones((C, D_HID), _f32)

    cnt = _count(dst_p, zeros, ones).reshape(NC, N_P, D_HID)
    h1 = _matmul1(x_p, W1)
    h1s, dinv = _scale1(cnt, h1)
    s1 = _seg_sum(h1s, src_p, dst_p, zeros).reshape(NC, N_P, D_HID)
    hs = _scale2(s1, h1s, dinv, b1.reshape(1, D_HID))
    s2 = _seg_sum(hs, src_p, dst_p, zeros).reshape(NC, N_P, D_HID)
    out = _final(s2, hs, dinv, W2, b2.reshape(1, D_OUT))
    return out[:N]


# speedup vs baseline: 39.9054x; 1.8322x over previous
import functools

import jax
import jax.numpy as jnp
from jax import lax
from jax.experimental import pallas as pl
from jax.experimental.pallas import tpu as pltpu
from jax.experimental.pallas import tpu_sc as plsc

N = 10000
E = 320000
D_IN = 128
D_HID = 16
D_OUT = 128

NC = 2
NS = 16
NW = NC * NS
C = 128
NCHUNK = 80
EPT = NCHUNK * C
E_PAD = EPT * NW
N_P = 10240
RPT = N_P // NS

_f32 = jnp.float32
_i32 = jnp.int32

_mesh = plsc.VectorSubcoreMesh(core_axis_name="c", subcore_axis_name="s")
_sc_params = pltpu.CompilerParams(use_tc_tiling_on_sc=False)


@functools.partial(
    pl.kernel,
    out_type=jax.ShapeDtypeStruct((NC * N_P, D_HID), _f32),
    mesh=_mesh,
    scratch_types=[
        pltpu.VMEM((NCHUNK, C), _i32),
        pltpu.VMEM((NCHUNK, C), _i32),
        pltpu.VMEM((C, D_HID), _f32),
        pltpu.VMEM((C, D_HID), _f32),
        pltpu.VMEM_SHARED((N_P, D_HID), _f32),
        pltpu.SemaphoreType.DMA,
        pltpu.SemaphoreType.DMA,
    ],
    compiler_params=_sc_params,
)
def _seg_sum(table_hbm, src_hbm, dst_hbm, zeros_hbm, out_hbm,
             src_v, dst_v, rows0, rows1, acc_sh, sem0, sem1):
    cid = lax.axis_index("c")
    sid = lax.axis_index("s")
    tid = cid * NS + sid
    pltpu.sync_copy(zeros_hbm.at[pl.ds(sid * RPT, RPT)],
                    acc_sh.at[pl.ds(sid * RPT, RPT)])
    pltpu.sync_copy(src_hbm.at[tid], src_v)
    pltpu.sync_copy(dst_hbm.at[tid], dst_v)
    plsc.subcore_barrier()

    pltpu.async_copy(table_hbm.at[src_v.at[0]], rows0, sem0)
    pltpu.async_copy(table_hbm.at[src_v.at[1]], rows1, sem1)

    @pl.loop(0, NCHUNK, step=2)
    def _(ci):
        pltpu.make_async_copy(table_hbm.at[src_v.at[ci]], rows0, sem0).wait()
        pltpu.sync_copy(rows0, acc_sh.at[dst_v.at[ci]], add=True)

        @pl.when(ci + 2 < NCHUNK)
        def _():
            pltpu.async_copy(table_hbm.at[src_v.at[ci + 2]], rows0, sem0)

        pltpu.make_async_copy(table_hbm.at[src_v.at[ci + 1]], rows1, sem1).wait()
        pltpu.sync_copy(rows1, acc_sh.at[dst_v.at[ci + 1]], add=True)

        @pl.when(ci + 3 < NCHUNK)
        def _():
            pltpu.async_copy(table_hbm.at[src_v.at[ci + 3]], rows1, sem1)

    plsc.subcore_barrier()
    pltpu.sync_copy(acc_sh.at[pl.ds(sid * RPT, RPT)],
                    out_hbm.at[pl.ds(cid * N_P + sid * RPT, RPT)])


@functools.partial(
    pl.kernel,
    out_type=jax.ShapeDtypeStruct((NC * N_P, D_HID), _f32),
    mesh=_mesh,
    scratch_types=[
        pltpu.VMEM((NCHUNK, C), _i32),
        pltpu.VMEM((C, D_HID), _f32),
        pltpu.VMEM_SHARED((N_P, D_HID), _f32),
    ],
    compiler_params=_sc_params,
)
def _count(dst_hbm, zeros_hbm, ones_hbm, out_hbm, dst_v, rows_v, acc_sh):
    cid = lax.axis_index("c")
    sid = lax.axis_index("s")
    tid = cid * NS + sid
    pltpu.sync_copy(zeros_hbm.at[pl.ds(sid * RPT, RPT)],
                    acc_sh.at[pl.ds(sid * RPT, RPT)])
    pltpu.sync_copy(dst_hbm.at[tid], dst_v)
    pltpu.sync_copy(ones_hbm, rows_v)
    plsc.subcore_barrier()

    @pl.loop(0, NCHUNK)
    def _(ci):
        pltpu.sync_copy(rows_v, acc_sh.at[dst_v.at[ci]], add=True)

    plsc.subcore_barrier()
    pltpu.sync_copy(acc_sh.at[pl.ds(sid * RPT, RPT)],
                    out_hbm.at[pl.ds(cid * N_P + sid * RPT, RPT)])


BLK = 1024
GRID = N_P // BLK


def _mm1_body(x_ref, w_ref, o_ref):
    o_ref[...] = jnp.dot(x_ref[...], w_ref[...], preferred_element_type=_f32)


def _matmul1(x_p, W1):
    return pl.pallas_call(
        _mm1_body,
        grid=(GRID,),
        in_specs=[pl.BlockSpec((BLK, D_IN), lambda i: (i, 0)),
                  pl.BlockSpec((D_IN, D_HID), lambda i: (0, 0))],
        out_specs=pl.BlockSpec((BLK, D_HID), lambda i: (i, 0)),
        out_shape=jax.ShapeDtypeStruct((N_P, D_HID), _f32),
    )(x_p, W1)


def _scale1_body(cnt_ref, h1_ref, hs_ref, dinv_ref):
    dinv = lax.rsqrt(cnt_ref[0] + cnt_ref[1] + 1.0)
    dinv_ref[...] = dinv
    hs_ref[...] = dinv * h1_ref[...]


def _scale1(cnt, h1):
    return pl.pallas_call(
        _scale1_body,
        grid=(GRID,),
        in_specs=[pl.BlockSpec((NC, BLK, D_HID), lambda i: (0, i, 0)),
                  pl.BlockSpec((BLK, D_HID), lambda i: (i, 0))],
        out_specs=[pl.BlockSpec((BLK, D_HID), lambda i: (i, 0)),
                   pl.BlockSpec((BLK, D_HID), lambda i: (i, 0))],
        out_shape=[jax.ShapeDtypeStruct((N_P, D_HID), _f32),
                   jax.ShapeDtypeStruct((N_P, D_HID), _f32)],
    )(cnt, h1)


def _scale2_body(s1_ref, h1s_ref, dinv_ref, b1_ref, o_ref):
    out1 = dinv_ref[...] * (s1_ref[0] + s1_ref[1] + h1s_ref[...]) + b1_ref[...]
    o_ref[...] = dinv_ref[...] * jnp.maximum(out1, 0.0)


def _scale2(s1, h1s, dinv, b1):
    return pl.pallas_call(
        _scale2_body,
        grid=(GRID,),
        in_specs=[pl.BlockSpec((NC, BLK, D_HID), lambda i: (0, i, 0)),
                  pl.BlockSpec((BLK, D_HID), lambda i: (i, 0)),
                  pl.BlockSpec((BLK, D_HID), lambda i: (i, 0)),
                  pl.BlockSpec((1, D_HID), lambda i: (0, 0))],
        out_specs=pl.BlockSpec((BLK, D_HID), lambda i: (i, 0)),
        out_shape=jax.ShapeDtypeStruct((N_P, D_HID), _f32),
    )(s1, h1s, dinv, b1)


def _final_body(s2_ref, hs_ref, dinv_ref, w2_ref, b2_ref, o_ref):
    u = dinv_ref[...] * (s2_ref[0] + s2_ref[1] + hs_ref[...])
    o_ref[...] = jnp.dot(u, w2_ref[...], preferred_element_type=_f32) + b2_ref[...]


def _final(s2, hs, dinv, W2, b2):
    return pl.pallas_call(
        _final_body,
        grid=(GRID,),
        in_specs=[pl.BlockSpec((NC, BLK, D_HID), lambda i: (0, i, 0)),
                  pl.BlockSpec((BLK, D_HID), lambda i: (i, 0)),
                  pl.BlockSpec((BLK, D_HID), lambda i: (i, 0)),
                  pl.BlockSpec((D_HID, D_OUT), lambda i: (0, 0)),
                  pl.BlockSpec((1, D_OUT), lambda i: (0, 0))],
        out_specs=pl.BlockSpec((BLK, D_OUT), lambda i: (i, 0)),
        out_shape=jax.ShapeDtypeStruct((N_P, D_OUT), _f32),
    )(s2, hs, dinv, W2, b2)


def kernel(x, edge_index, W1, b1, W2, b2):
    x = x.astype(_f32)
    src = edge_index[0].astype(_i32)
    dst = edge_index[1].astype(_i32)
    src_p = jnp.concatenate([src, jnp.zeros((E_PAD - E,), _i32)])
    dst_p = jnp.concatenate([dst, jnp.full((E_PAD - E,), N, _i32)])
    src_p = src_p.reshape(NW, NCHUNK, C)
    dst_p = dst_p.reshape(NW, NCHUNK, C)
    x_p = jnp.pad(x, ((0, N_P - N), (0, 0)))
    zeros = jnp.zeros((N_P, D_HID), _f32)
    ones = jnp.ones((C, D_HID), _f32)

    cnt = _count(dst_p, zeros, ones).reshape(NC, N_P, D_HID)
    h1 = _matmul1(x_p, W1)
    h1s, dinv = _scale1(cnt, h1)
    s1 = _seg_sum(h1s, src_p, dst_p, zeros).reshape(NC, N_P, D_HID)
    hs = _scale2(s1, h1s, dinv, b1.reshape(1, D_HID))
    s2 = _seg_sum(hs, src_p, dst_p, zeros).reshape(NC, N_P, D_HID)
    out = _final(s2, hs, dinv, W2, b2.reshape(1, D_OUT))
    return out[:N]


# speedup vs baseline: 54.1400x; 1.3567x over previous
import functools

import jax
import jax.numpy as jnp
from jax import lax
from jax.experimental import pallas as pl
from jax.experimental.pallas import tpu as pltpu
from jax.experimental.pallas import tpu_sc as plsc

N = 10000
E = 320000
D_IN = 128
D_HID = 16
D_OUT = 128

NC = 2
NS = 16
NW = NC * NS
C = 128
NCHUNK = 80
EPT = NCHUNK * C
E_PAD = EPT * NW
N_P = 10240
RPT = N_P // NS
G = 8
NF = N_P // G
W128 = G * D_HID

_f32 = jnp.float32
_i32 = jnp.int32

_mesh = plsc.VectorSubcoreMesh(core_axis_name="c", subcore_axis_name="s",
                               num_cores=NC, num_subcores=NS)
_sc_params = pltpu.CompilerParams(use_tc_tiling_on_sc=False)


@functools.partial(
    pl.kernel,
    out_type=jax.ShapeDtypeStruct((NC * N_P, D_HID), _f32),
    mesh=_mesh,
    scratch_types=[
        pltpu.VMEM((NCHUNK, C), _i32),
        pltpu.VMEM((NCHUNK, C), _i32),
        pltpu.VMEM((C, D_HID), _f32),
        pltpu.VMEM((C, D_HID), _f32),
        pltpu.VMEM_SHARED((N_P, D_HID), _f32),
        pltpu.SemaphoreType.DMA,
        pltpu.SemaphoreType.DMA,
    ],
    compiler_params=_sc_params,
)
def _seg_sum(table_hbm, src_hbm, dst_hbm, out_hbm,
             src_v, dst_v, rows0, rows1, acc_sh, sem0, sem1):
    cid = lax.axis_index("c")
    sid = lax.axis_index("s")
    tid = cid * NS + sid

    @pl.loop(0, C)
    def _(i):
        rows0[i, :] = jnp.zeros((D_HID,), _f32)

    @pl.loop(0, RPT // C)
    def _(k):
        pltpu.sync_copy(rows0, acc_sh.at[pl.ds(sid * RPT + k * C, C)])

    pltpu.sync_copy(src_hbm.at[tid], src_v)
    pltpu.sync_copy(dst_hbm.at[tid], dst_v)
    plsc.subcore_barrier()

    pltpu.async_copy(table_hbm.at[src_v.at[0]], rows0, sem0)
    pltpu.async_copy(table_hbm.at[src_v.at[1]], rows1, sem1)

    @pl.loop(0, NCHUNK, step=2)
    def _(ci):
        pltpu.make_async_copy(table_hbm.at[src_v.at[ci]], rows0, sem0).wait()
        pltpu.sync_copy(rows0, acc_sh.at[dst_v.at[ci]], add=True)

        @pl.when(ci + 2 < NCHUNK)
        def _():
            pltpu.async_copy(table_hbm.at[src_v.at[ci + 2]], rows0, sem0)

        pltpu.make_async_copy(table_hbm.at[src_v.at[ci + 1]], rows1, sem1).wait()
        pltpu.sync_copy(rows1, acc_sh.at[dst_v.at[ci + 1]], add=True)

        @pl.when(ci + 3 < NCHUNK)
        def _():
            pltpu.async_copy(table_hbm.at[src_v.at[ci + 3]], rows1, sem1)

    plsc.subcore_barrier()
    pltpu.sync_copy(acc_sh.at[pl.ds(sid * RPT, RPT)],
                    out_hbm.at[pl.ds(cid * N_P + sid * RPT, RPT)])


@functools.partial(
    pl.kernel,
    out_type=jax.ShapeDtypeStruct((NC * N_P, D_HID), _f32),
    mesh=_mesh,
    scratch_types=[
        pltpu.VMEM((NCHUNK, C), _i32),
        pltpu.VMEM((C, D_HID), _f32),
        pltpu.VMEM_SHARED((N_P, D_HID), _f32),
    ],
    compiler_params=_sc_params,
)
def _count(dst_hbm, out_hbm, dst_v, rows_v, acc_sh):
    cid = lax.axis_index("c")
    sid = lax.axis_index("s")
    tid = cid * NS + sid

    @pl.loop(0, C)
    def _(i):
        rows_v[i, :] = jnp.zeros((D_HID,), _f32)

    @pl.loop(0, RPT // C)
    def _(k):
        pltpu.sync_copy(rows_v, acc_sh.at[pl.ds(sid * RPT + k * C, C)])

    @pl.loop(0, C)
    def _(i):
        rows_v[i, :] = jnp.ones((D_HID,), _f32)

    pltpu.sync_copy(dst_hbm.at[tid], dst_v)
    plsc.subcore_barrier()

    @pl.loop(0, NCHUNK)
    def _(ci):
        pltpu.sync_copy(rows_v, acc_sh.at[dst_v.at[ci]], add=True)

    plsc.subcore_barrier()
    pltpu.sync_copy(acc_sh.at[pl.ds(sid * RPT, RPT)],
                    out_hbm.at[pl.ds(cid * N_P + sid * RPT, RPT)])


def _full(shape):
    return pl.BlockSpec(shape, lambda i: tuple(0 for _ in shape))


def _p2():
    return [pl.BlockSpec((NF, W128), lambda i: (0, 0)),
            pl.BlockSpec((NF, W128), lambda i: (1, 0))]


def _mm1_body(xg_ref, wbd_ref, o_ref):
    o_ref[...] = jnp.dot(xg_ref[...], wbd_ref[...], preferred_element_type=_f32)


def _matmul1(x_g, W_bd):
    return pl.pallas_call(
        _mm1_body,
        grid=(1,),
        in_specs=[_full((NF, G * D_IN)), _full((G * D_IN, W128))],
        out_specs=_full((NF, W128)),
        out_shape=jax.ShapeDtypeStruct((NF, W128), _f32),
    )(x_g, W_bd)


def _scale1_body(cnt0_ref, cnt1_ref, h1_ref, hs_ref, dinv_ref):
    dinv = lax.rsqrt(cnt0_ref[...] + cnt1_ref[...] + 1.0)
    dinv_ref[...] = dinv
    hs_ref[...] = dinv * h1_ref[...]


def _scale1(cnt_f, h1_f):
    return pl.pallas_call(
        _scale1_body,
        grid=(1,),
        in_specs=_p2() + [_full((NF, W128))],
        out_specs=[_full((NF, W128)), _full((NF, W128))],
        out_shape=[jax.ShapeDtypeStruct((NF, W128), _f32),
                   jax.ShapeDtypeStruct((NF, W128), _f32)],
    )(cnt_f, cnt_f, h1_f)


def _scale2_body(s10_ref, s11_ref, h1s_ref, dinv_ref, b1_ref, o_ref):
    out1 = (dinv_ref[...] * (s10_ref[...] + s11_ref[...] + h1s_ref[...])
            + b1_ref[...])
    o_ref[...] = dinv_ref[...] * jnp.maximum(out1, 0.0)


def _scale2(s1_f, h1s_f, dinv_f, b1_t):
    return pl.pallas_call(
        _scale2_body,
        grid=(1,),
        in_specs=_p2() + [_full((NF, W128)), _full((NF, W128)),
                          _full((1, W128))],
        out_specs=_full((NF, W128)),
        out_shape=jax.ShapeDtypeStruct((NF, W128), _f32),
    )(s1_f, s1_f, h1s_f, dinv_f, b1_t)


def _final_body(s20_ref, s21_ref, hs_ref, dinv_ref, w2bd_ref, b2_ref, o_ref):
    u = dinv_ref[...] * (s20_ref[...] + s21_ref[...] + hs_ref[...])
    o_ref[...] = (jnp.dot(u, w2bd_ref[...], preferred_element_type=_f32)
                  + b2_ref[...])


def _final(s2_f, hs_f, dinv_f, W2_bd, b2_t):
    return pl.pallas_call(
        _final_body,
        grid=(1,),
        in_specs=_p2() + [_full((NF, W128)), _full((NF, W128)),
                          _full((W128, G * D_OUT)), _full((1, G * D_OUT))],
        out_specs=_full((NF, G * D_OUT)),
        out_shape=jax.ShapeDtypeStruct((NF, G * D_OUT), _f32),
    )(s2_f, s2_f, hs_f, dinv_f, W2_bd, b2_t)


def _block_diag(W, g):
    a, b = W.shape
    eye = jnp.eye(g, dtype=W.dtype)
    return (eye[:, None, :, None] * W[None, :, None, :]).reshape(g * a, g * b)


def kernel(x, edge_index, W1, b1, W2, b2):
    x = x.astype(_f32)
    src = edge_index[0].astype(_i32)
    dst = edge_index[1].astype(_i32)
    src_p = jnp.concatenate([src, jnp.zeros((E_PAD - E,), _i32)])
    dst_p = jnp.concatenate([dst, jnp.full((E_PAD - E,), N, _i32)])
    src_p = src_p.reshape(NW, NCHUNK, C)
    dst_p = dst_p.reshape(NW, NCHUNK, C)
    x_g = jnp.pad(x, ((0, N_P - N), (0, 0))).reshape(NF, G * D_IN)
    W_bd = _block_diag(W1, G)
    W2_bd = _block_diag(W2, G)
    b1_t = jnp.tile(b1, G).reshape(1, W128)
    b2_t = jnp.tile(b2, G).reshape(1, G * D_OUT)

    cnt = _count(dst_p)
    cnt_f = cnt.reshape(2 * NF, W128)
    h1_f = _matmul1(x_g, W_bd)
    h1s_f, dinv_f = _scale1(cnt_f, h1_f)
    s1 = _seg_sum(h1s_f.reshape(N_P, D_HID), src_p, dst_p)
    hs_f = _scale2(s1.reshape(2 * NF, W128), h1s_f, dinv_f, b1_t)
    s2 = _seg_sum(hs_f.reshape(N_P, D_HID), src_p, dst_p)
    out_g = _final(s2.reshape(2 * NF, W128), hs_f, dinv_f, W2_bd, b2_t)
    return out_g.reshape(N_P, D_OUT)[:N]
